# baseline (speedup 1.0000x reference)
import jax
import jax.numpy as jnp
from jax.experimental import pallas as pl

N = 10000


def _add_bias_kernel(h_ref, b_ref, o_ref):
    o_ref[...] = h_ref[...] + b_ref[...]


def _gcn_conv(x, src, dst, norm, W, b):
    h = x @ W
    msgs = h[src] * norm[:, None]
    out = jnp.zeros((N, h.shape[1]), dtype=h.dtype).at[dst].add(msgs)
    return pl.pallas_call(
        _add_bias_kernel,
        out_shape=jax.ShapeDtypeStruct(out.shape, out.dtype),
    )(out, jnp.broadcast_to(b[None, :], out.shape))


def kernel(x, edge_index, W1, b1, W2, b2):
    src = edge_index[0]
    dst = edge_index[1]
    loop = jnp.arange(N, dtype=src.dtype)
    src = jnp.concatenate([src, loop])
    dst = jnp.concatenate([dst, loop])
    deg = jnp.zeros((N,), dtype=jnp.float32).at[dst].add(1.0)
    deg_inv_sqrt = jnp.where(deg > 0, deg ** -0.5, 0.0)
    norm = deg_inv_sqrt[src] * deg_inv_sqrt[dst]
    h = _gcn_conv(x, src, dst, norm, W1, b1)
    h = jax.nn.relu(h)
    out = _gcn_conv(h, src, dst, norm, W2, b2)
    return out


# speedup vs baseline: 8.4120x; 8.4120x over previous
import jax
import jax.numpy as jnp
from jax import lax
from jax.experimental import pallas as pl
from jax.experimental.pallas import tpu as pltpu
from jax.experimental.pallas import tpu_sc as plsc

N = 10000
D = 256
E = 160000

NC = 2
NS = 16
HALF = N // NC
ACC_ROWS = 5120
TRASH = HALF
CHUNK = 80
EPW = E // NS
NCHUNK = EPW // CHUNK
DEG_ROWS = 80

ROWBLK = 200
_HIGHEST = lax.Precision.HIGHEST

import dataclasses as _dataclasses
import functools as _functools


@_functools.cache
def _sc_compiler_params():
    cp = pltpu.CompilerParams()
    if "needs_layout_passes" in pltpu.CompilerParams.__dataclass_fields__:
        cp = _dataclasses.replace(cp, needs_layout_passes=False)
    return cp


@_functools.cache
def _vector_mesh():
    return plsc.VectorSubcoreMesh(core_axis_name="core",
                                  subcore_axis_name="subcore",
                                  num_cores=NC, num_subcores=NS)


def _deg_body(dst_hbm, iota_hbm, deg_hbm, dstv, part, iotav, spdeg):
    c = lax.axis_index("core")
    s = lax.axis_index("subcore")
    stripe = 8

    @pl.loop(0, DEG_ROWS)
    def _(i):
        for j in range(8):
            part[i, pl.ds(j * 16, 16)] = jnp.zeros((16,), jnp.float32)

    @pl.when(s < DEG_ROWS // stripe)
    def _():
        pltpu.sync_copy(part.at[pl.ds(0, stripe)],
                        spdeg.at[pl.ds(s * stripe, stripe)])

    pltpu.sync_copy(iota_hbm, iotav)
    pltpu.sync_copy(dst_hbm.at[pl.ds(s * EPW, EPW)], dstv)

    @pl.loop(0, EPW // 16)
    def _(i):
        idx = dstv[pl.ds(i * 16, 16)]
        row = lax.shift_right_logical(idx, 7)
        col = jnp.bitwise_and(idx, 127)
        plsc.addupdate_scatter(part, [row, col], jnp.ones((16,), jnp.float32))

    plsc.subcore_barrier()
    pltpu.sync_copy(part, spdeg.at[iotav], add=True)
    plsc.subcore_barrier()

    @pl.when((c == 0) & (s < DEG_ROWS // stripe))
    def _():
        pltpu.sync_copy(spdeg.at[pl.ds(s * stripe, stripe)],
                        deg_hbm.at[pl.ds(s * stripe, stripe)])


def _compute_deg(dst, iota80):
    kfn = pl.kernel(
        _deg_body,
        out_type=jax.ShapeDtypeStruct((DEG_ROWS, 128), jnp.float32),
        mesh=_vector_mesh(),
        compiler_params=_sc_compiler_params(),
        scratch_types=[
            pltpu.VMEM((EPW,), jnp.int32),
            pltpu.VMEM((DEG_ROWS, 128), jnp.float32),
            pltpu.VMEM((DEG_ROWS,), jnp.int32),
            pltpu.VMEM_SHARED((DEG_ROWS, 128), jnp.float32),
        ],
    )
    return kfn(dst, iota80)


def _agg_body(src_hbm, dst2d_hbm, hs2_hbm, out_hbm,
              srcv, dstloc, rowsA, rowsB, zblk, acc, semA, semB):
    c = lax.axis_index("core")
    s = lax.axis_index("subcore")

    pltpu.sync_copy(src_hbm.at[pl.ds(s * EPW, EPW)], srcv)

    @pl.loop(0, EPW // 16)
    def _(i):
        v = srcv[pl.ds(i * 16, 16)]
        srcv[pl.ds(i * 16, 16)] = v + v + c

    @pl.loop(0, 64)
    def _(i):
        for j in range(128 // 16):
            zblk[i, pl.ds(j * 16, 16)] = jnp.zeros((16,), jnp.float32)

    def start_gather(i, buf, sem):
        pltpu.async_copy(hs2_hbm.at[srcv.at[pl.ds(i * CHUNK, CHUNK)]], buf, sem)

    def wait_gather(i, buf, sem):
        pltpu.make_async_copy(hs2_hbm.at[srcv.at[pl.ds(i * CHUNK, CHUNK)]],
                              buf, sem).wait()

    for p in range(2):
        for k in range(5):
            pltpu.sync_copy(zblk, acc.at[pl.ds(s * 320 + k * 64, 64)])

        pltpu.sync_copy(dst2d_hbm.at[s], dstloc)
        base = p * HALF

        @pl.loop(0, NCHUNK)
        def _(i):
            for j in range(CHUNK // 16):
                d = dstloc[i, pl.ds(j * 16, 16)]
                local = d - base
                ok = (local >= 0) & (local < HALF)
                dstloc[i, pl.ds(j * 16, 16)] = jnp.where(ok, local, TRASH)

        plsc.subcore_barrier()

        start_gather(0, rowsA, semA)

        @pl.loop(0, NCHUNK // 2)
        def _(it):
            i = it * 2
            start_gather(i + 1, rowsB, semB)
            wait_gather(i, rowsA, semA)
            pltpu.sync_copy(rowsA, acc.at[dstloc.at[i]], add=True)

            @pl.when(i + 2 < NCHUNK)
            def _():
                start_gather(i + 2, rowsA, semA)

            wait_gather(i + 1, rowsB, semB)
            pltpu.sync_copy(rowsB, acc.at[dstloc.at[i + 1]], add=True)

        if NCHUNK % 2 == 1:
            wait_gather(NCHUNK - 1, rowsA, semA)
            pltpu.sync_copy(rowsA, acc.at[dstloc.at[NCHUNK - 1]], add=True)

        plsc.subcore_barrier()

        pltpu.sync_copy(acc.at[pl.ds(s * 320, 320)],
                        out_hbm.at[c, p, pl.ds(s * 320, 320)])


def _aggregate(src, dst2d, hs):
    kfn = pl.kernel(
        _agg_body,
        out_type=jax.ShapeDtypeStruct((NC, 2, ACC_ROWS, 128), jnp.float32),
        mesh=_vector_mesh(),
        compiler_params=_sc_compiler_params(),
        scratch_types=[
            pltpu.VMEM((EPW,), jnp.int32),
            pltpu.VMEM((NCHUNK, CHUNK), jnp.int32),
            pltpu.VMEM((CHUNK, 128), jnp.float32),
            pltpu.VMEM((CHUNK, 128), jnp.float32),
            pltpu.VMEM((64, 128), jnp.float32),
            pltpu.VMEM_SHARED((ACC_ROWS, 128), jnp.float32),
            pltpu.SemaphoreType.DMA,
            pltpu.SemaphoreType.DMA,
        ],
    )
    return kfn(src, dst2d, hs.reshape(2 * N, 128))


def _k1_body(deg_ref, x_ref, w_ref, hs_ref, dis_ref):
    dis = lax.rsqrt(deg_ref[...] + 1.0)
    h = jnp.dot(x_ref[...], w_ref[...], precision=_HIGHEST,
                preferred_element_type=jnp.float32)
    hs_ref[...] = h * dis
    dis_ref[...] = dis


def _k2_body(a0_ref, a1_ref, hs_ref, dis_ref, b_ref, w_ref, hs2_ref):
    dis = dis_ref[...]
    agg = jnp.concatenate([a0_ref[0], a1_ref[0]], axis=1)
    t = jnp.maximum(dis * (agg + hs_ref[...]) + b_ref[...], 0.0)
    hs2_ref[...] = dis * jnp.dot(t, w_ref[...], precision=_HIGHEST,
                                 preferred_element_type=jnp.float32)


def _k3_body(a0_ref, a1_ref, hs_ref, dis_ref, b_ref, o_ref):
    agg = jnp.concatenate([a0_ref[0], a1_ref[0]], axis=1)
    o_ref[...] = dis_ref[...] * (agg + hs_ref[...]) + b_ref[...]


_NBLK = N // ROWBLK
_BPH = HALF // ROWBLK

def _row_spec(width):
    return pl.BlockSpec((ROWBLK, width), lambda i: (i, 0))

def _agg_spec():
    return pl.BlockSpec((1, ROWBLK, 128), lambda i: (i // _BPH, i % _BPH, 0))

def _full_spec(r, cols):
    return pl.BlockSpec((r, cols), lambda i: (0, 0))


def _tc_k1(deg, x, W1):
    return pl.pallas_call(
        _k1_body,
        grid=(_NBLK,),
        in_specs=[_row_spec(1), _row_spec(D), _full_spec(D, D)],
        out_specs=[_row_spec(D), _row_spec(1)],
        out_shape=[jax.ShapeDtypeStruct((N, D), jnp.float32),
                   jax.ShapeDtypeStruct((N, 1), jnp.float32)],
    )(deg, x, W1)


def _tc_k2(agg, hs, dis, b1, W2):
    return pl.pallas_call(
        _k2_body,
        grid=(_NBLK,),
        in_specs=[_agg_spec(), _agg_spec(), _row_spec(D), _row_spec(1),
                  _full_spec(1, D), _full_spec(D, D)],
        out_specs=_row_spec(D),
        out_shape=jax.ShapeDtypeStruct((N, D), jnp.float32),
    )(agg[0], agg[1], hs, dis, b1, W2)


def _tc_k3(agg, hs, dis, b2):
    return pl.pallas_call(
        _k3_body,
        grid=(_NBLK,),
        in_specs=[_agg_spec(), _agg_spec(), _row_spec(D), _row_spec(1),
                  _full_spec(1, D)],
        out_specs=_row_spec(D),
        out_shape=jax.ShapeDtypeStruct((N, D), jnp.float32),
    )(agg[0], agg[1], hs, dis, b2)


def kernel(x, edge_index, W1, b1, W2, b2):
    src = edge_index[0].astype(jnp.int32)
    dst = edge_index[1].astype(jnp.int32)
    dst2d = dst.reshape(NS, NCHUNK, CHUNK)
    iota80 = jnp.arange(DEG_ROWS, dtype=jnp.int32)
    b1r = b1.reshape(1, D)
    b2r = b2.reshape(1, D)

    deg = _compute_deg(dst, iota80).reshape(-1)[:N].reshape(N, 1)
    hs1, dis = _tc_k1(deg, x, W1)
    agg1 = _aggregate(src, dst2d, hs1)
    hs2 = _tc_k2(agg1, hs1, dis, b1r, W2)
    agg2 = _aggregate(src, dst2d, hs2)
    return _tc_k3(agg2, hs2, dis, b2r)


# speedup vs baseline: 10.7627x; 1.2795x over previous
import jax
import jax.numpy as jnp
from jax import lax
from jax.experimental import pallas as pl
from jax.experimental.pallas import tpu as pltpu
from jax.experimental.pallas import tpu_sc as plsc

N = 10000
D = 256
E = 160000

NC = 2
NS = 16
ACC_ROWS = 10240
CHUNK = 80
EPW = E // NS
NCHUNK = EPW // CHUNK
DEG_ROWS = 80

ROWBLK = 80
_HIGHEST = lax.Precision.HIGHEST

import dataclasses as _dataclasses
import functools as _functools


@_functools.cache
def _sc_compiler_params():
    cp = pltpu.CompilerParams()
    if "needs_layout_passes" in pltpu.CompilerParams.__dataclass_fields__:
        cp = _dataclasses.replace(cp, needs_layout_passes=False)
    return cp


@_functools.cache
def _vector_mesh():
    return plsc.VectorSubcoreMesh(core_axis_name="core",
                                  subcore_axis_name="subcore",
                                  num_cores=NC, num_subcores=NS)


def _deg_body(dst_hbm, iota_hbm, deg_hbm, dstv, part, iotav, spdeg):
    c = lax.axis_index("core")
    s = lax.axis_index("subcore")
    stripe = 8

    @pl.loop(0, DEG_ROWS)
    def _(i):
        for j in range(8):
            part[i, pl.ds(j * 16, 16)] = jnp.zeros((16,), jnp.float32)

    @pl.when(s < DEG_ROWS // stripe)
    def _():
        pltpu.sync_copy(part.at[pl.ds(0, stripe)],
                        spdeg.at[pl.ds(s * stripe, stripe)])

    pltpu.sync_copy(iota_hbm, iotav)
    pltpu.sync_copy(dst_hbm.at[pl.ds(s * EPW, EPW)], dstv)

    @pl.loop(0, EPW // 16)
    def _(i):
        idx = dstv[pl.ds(i * 16, 16)]
        row = lax.shift_right_logical(idx, 7)
        col = jnp.bitwise_and(idx, 127)
        plsc.addupdate_scatter(part, [row, col], jnp.ones((16,), jnp.float32))

    plsc.subcore_barrier()
    pltpu.sync_copy(part, spdeg.at[iotav], add=True)
    plsc.subcore_barrier()

    @pl.when((c == 0) & (s < DEG_ROWS // stripe))
    def _():
        pltpu.sync_copy(spdeg.at[pl.ds(s * stripe, stripe)],
                        deg_hbm.at[pl.ds(s * stripe, stripe)])


def _compute_deg(dst, iota80):
    kfn = pl.kernel(
        _deg_body,
        out_type=jax.ShapeDtypeStruct((DEG_ROWS, 128), jnp.float32),
        mesh=_vector_mesh(),
        compiler_params=_sc_compiler_params(),
        scratch_types=[
            pltpu.VMEM((EPW,), jnp.int32),
            pltpu.VMEM((DEG_ROWS, 128), jnp.float32),
            pltpu.VMEM((DEG_ROWS,), jnp.int32),
            pltpu.VMEM_SHARED((DEG_ROWS, 128), jnp.float32),
        ],
    )
    return kfn(dst, iota80)


def _agg_body(src_hbm, dst2d_hbm, hs2_hbm, out_hbm,
              srcv, dstloc, rowsA, rowsB, zblk, acc, semA, semB):
    c = lax.axis_index("core")
    s = lax.axis_index("subcore")

    pltpu.sync_copy(src_hbm.at[pl.ds(s * EPW, EPW)], srcv)

    @pl.loop(0, EPW // 16)
    def _(i):
        v = srcv[pl.ds(i * 16, 16)]
        srcv[pl.ds(i * 16, 16)] = v + v + c

    @pl.loop(0, 8)
    def _(i):
        for j in range(128 // 16):
            zblk[i, pl.ds(j * 16, 16)] = jnp.zeros((16,), jnp.float32)

    def start_gather(i, buf, sem):
        pltpu.async_copy(hs2_hbm.at[srcv.at[pl.ds(i * CHUNK, CHUNK)]], buf, sem)

    def wait_gather(i, buf, sem):
        pltpu.make_async_copy(hs2_hbm.at[srcv.at[pl.ds(i * CHUNK, CHUNK)]],
                              buf, sem).wait()

    @pl.loop(0, 80)
    def _(k):
        pltpu.sync_copy(zblk, acc.at[pl.ds(s * 640 + k * 8, 8)])

    pltpu.sync_copy(dst2d_hbm.at[s], dstloc)
    plsc.subcore_barrier()

    start_gather(0, rowsA, semA)

    @pl.loop(0, NCHUNK // 2)
    def _(it):
        i = it * 2
        start_gather(i + 1, rowsB, semB)
        wait_gather(i, rowsA, semA)
        pltpu.sync_copy(rowsA, acc.at[dstloc.at[i]], add=True)

        @pl.when(i + 2 < NCHUNK)
        def _():
            start_gather(i + 2, rowsA, semA)

        wait_gather(i + 1, rowsB, semB)
        pltpu.sync_copy(rowsB, acc.at[dstloc.at[i + 1]], add=True)

    if NCHUNK % 2 == 1:
        wait_gather(NCHUNK - 1, rowsA, semA)
        pltpu.sync_copy(rowsA, acc.at[dstloc.at[NCHUNK - 1]], add=True)

    plsc.subcore_barrier()

    pltpu.sync_copy(acc.at[pl.ds(s * 640, 640)],
                    out_hbm.at[c, pl.ds(s * 640, 640)])


def _aggregate(src, dst2d, hs):
    kfn = pl.kernel(
        _agg_body,
        out_type=jax.ShapeDtypeStruct((NC, ACC_ROWS, 128), jnp.float32),
        mesh=_vector_mesh(),
        compiler_params=_sc_compiler_params(),
        scratch_types=[
            pltpu.VMEM((EPW,), jnp.int32),
            pltpu.VMEM((NCHUNK, CHUNK), jnp.int32),
            pltpu.VMEM((CHUNK, 128), jnp.float32),
            pltpu.VMEM((CHUNK, 128), jnp.float32),
            pltpu.VMEM((8, 128), jnp.float32),
            pltpu.VMEM_SHARED((ACC_ROWS, 128), jnp.float32),
            pltpu.SemaphoreType.DMA,
            pltpu.SemaphoreType.DMA,
        ],
    )
    return kfn(src, dst2d, hs.reshape(2 * N, 128))


def _k1_body(deg_ref, x_ref, w_ref, hs_ref, dis_ref):
    dis = lax.rsqrt(deg_ref[...] + 1.0)
    h = jnp.dot(x_ref[...], w_ref[...], precision=_HIGHEST,
                preferred_element_type=jnp.float32)
    hs_ref[...] = h * dis
    dis_ref[...] = dis


def _k2_body(a0_ref, a1_ref, hs_ref, dis_ref, b_ref, w_ref, hs2_ref):
    dis = dis_ref[...]
    agg = jnp.concatenate([a0_ref[...], a1_ref[...]], axis=1)
    t = jnp.maximum(dis * (agg + hs_ref[...]) + b_ref[...], 0.0)
    hs2_ref[...] = dis * jnp.dot(t, w_ref[...], precision=_HIGHEST,
                                 preferred_element_type=jnp.float32)


def _k3_body(a0_ref, a1_ref, hs_ref, dis_ref, b_ref, o_ref):
    agg = jnp.concatenate([a0_ref[...], a1_ref[...]], axis=1)
    o_ref[...] = dis_ref[...] * (agg + hs_ref[...]) + b_ref[...]


_NBLK = N // ROWBLK

def _row_spec(width):
    return pl.BlockSpec((ROWBLK, width), lambda i: (i, 0))

def _agg_spec():
    return pl.BlockSpec((ROWBLK, 128), lambda i: (i, 0))

def _full_spec(r, cols):
    return pl.BlockSpec((r, cols), lambda i: (0, 0))


def _tc_k1(deg, x, W1):
    return pl.pallas_call(
        _k1_body,
        grid=(_NBLK,),
        in_specs=[_row_spec(1), _row_spec(D), _full_spec(D, D)],
        out_specs=[_row_spec(D), _row_spec(1)],
        out_shape=[jax.ShapeDtypeStruct((N, D), jnp.float32),
                   jax.ShapeDtypeStruct((N, 1), jnp.float32)],
    )(deg, x, W1)


def _tc_k2(agg, hs, dis, b1, W2):
    return pl.pallas_call(
        _k2_body,
        grid=(_NBLK,),
        in_specs=[_agg_spec(), _agg_spec(), _row_spec(D), _row_spec(1),
                  _full_spec(1, D), _full_spec(D, D)],
        out_specs=_row_spec(D),
        out_shape=jax.ShapeDtypeStruct((N, D), jnp.float32),
    )(agg[0], agg[1], hs, dis, b1, W2)


def _tc_k3(agg, hs, dis, b2):
    return pl.pallas_call(
        _k3_body,
        grid=(_NBLK,),
        in_specs=[_agg_spec(), _agg_spec(), _row_spec(D), _row_spec(1),
                  _full_spec(1, D)],
        out_specs=_row_spec(D),
        out_shape=jax.ShapeDtypeStruct((N, D), jnp.float32),
    )(agg[0], agg[1], hs, dis, b2)


def kernel(x, edge_index, W1, b1, W2, b2):
    src = edge_index[0].astype(jnp.int32)
    dst = edge_index[1].astype(jnp.int32)
    dst2d = dst.reshape(NS, NCHUNK, CHUNK)
    iota80 = jnp.arange(DEG_ROWS, dtype=jnp.int32)
    b1r = b1.reshape(1, D)
    b2r = b2.reshape(1, D)

    deg = _compute_deg(dst, iota80).reshape(-1)[:N].reshape(N, 1)
    hs1, dis = _tc_k1(deg, x, W1)
    agg1 = _aggregate(src, dst2d, hs1)
    hs2 = _tc_k2(agg1, hs1, dis, b1r, W2)
    agg2 = _aggregate(src, dst2d, hs2)
    return _tc_k3(agg2, hs2, dis, b2r)


# speedup vs baseline: 17.5944x; 1.6348x over previous
import jax
import jax.numpy as jnp
from jax import lax
from jax.experimental import pallas as pl
from jax.experimental.pallas import tpu as pltpu
from jax.experimental.pallas import tpu_sc as plsc

N = 10000
D = 256
E = 160000

NC = 2
NS = 16
ACC_ROWS = 10240
STRIPE = ACC_ROWS // NS
CHUNK = 80
EPW = E // NS
NCHUNK = EPW // CHUNK
DEG_ROWS = 80

ROWBLK = 1000

import dataclasses as _dataclasses
import functools as _functools


@_functools.cache
def _sc_compiler_params():
    cp = pltpu.CompilerParams()
    if "needs_layout_passes" in pltpu.CompilerParams.__dataclass_fields__:
        cp = _dataclasses.replace(cp, needs_layout_passes=False)
    return cp


@_functools.cache
def _vector_mesh():
    return plsc.VectorSubcoreMesh(core_axis_name="core",
                                  subcore_axis_name="subcore",
                                  num_cores=NC, num_subcores=NS)


def _deg_body(dst_hbm, iota_hbm, deg_hbm, dstv, part, iotav, spdeg):
    c = lax.axis_index("core")
    s = lax.axis_index("subcore")
    stripe = 8

    @pl.loop(0, DEG_ROWS)
    def _(i):
        for j in range(8):
            part[i, pl.ds(j * 16, 16)] = jnp.zeros((16,), jnp.float32)

    @pl.when(s < DEG_ROWS // stripe)
    def _():
        pltpu.sync_copy(part.at[pl.ds(0, stripe)],
                        spdeg.at[pl.ds(s * stripe, stripe)])

    pltpu.sync_copy(iota_hbm, iotav)
    pltpu.sync_copy(dst_hbm.at[pl.ds(s * EPW, EPW)], dstv)

    @pl.loop(0, EPW // 16)
    def _(i):
        idx = dstv[pl.ds(i * 16, 16)]
        row = lax.shift_right_logical(idx, 7)
        col = jnp.bitwise_and(idx, 127)
        plsc.addupdate_scatter(part, [row, col], jnp.ones((16,), jnp.float32))

    plsc.subcore_barrier()
    pltpu.sync_copy(part, spdeg.at[iotav], add=True)
    plsc.subcore_barrier()

    @pl.when((c == 0) & (s < DEG_ROWS // stripe))
    def _():
        pltpu.sync_copy(spdeg.at[pl.ds(s * stripe, stripe)],
                        deg_hbm.at[pl.ds(s * stripe, stripe)])


def _compute_deg(dst, iota80):
    kfn = pl.kernel(
        _deg_body,
        out_type=jax.ShapeDtypeStruct((DEG_ROWS, 128), jnp.float32),
        mesh=_vector_mesh(),
        compiler_params=_sc_compiler_params(),
        scratch_types=[
            pltpu.VMEM((EPW,), jnp.int32),
            pltpu.VMEM((DEG_ROWS, 128), jnp.float32),
            pltpu.VMEM((DEG_ROWS,), jnp.int32),
            pltpu.VMEM_SHARED((DEG_ROWS, 128), jnp.float32),
        ],
    )
    return kfn(dst, iota80)


def _agg_body(src_hbm, dst2d_hbm, hs2_hbm, out_hbm,
              srcv, dstloc, rowsA, rowsB, zblk, acc, semA, semB):
    c = lax.axis_index("core")
    s = lax.axis_index("subcore")

    pltpu.sync_copy(src_hbm.at[pl.ds(s * EPW, EPW)], srcv)

    @pl.loop(0, EPW // 16)
    def _(i):
        v = srcv[pl.ds(i * 16, 16)]
        srcv[pl.ds(i * 16, 16)] = v + v + c

    @pl.loop(0, 8)
    def _(i):
        for j in range(128 // 16):
            zblk[i, pl.ds(j * 16, 16)] = jnp.zeros((16,), jnp.float32)

    def start_gather(i, buf, sem):
        pltpu.async_copy(hs2_hbm.at[srcv.at[pl.ds(i * CHUNK, CHUNK)]], buf, sem)

    def wait_gather(i, buf, sem):
        pltpu.make_async_copy(hs2_hbm.at[srcv.at[pl.ds(i * CHUNK, CHUNK)]],
                              buf, sem).wait()

    @pl.loop(0, STRIPE // 8)
    def _(k):
        pltpu.sync_copy(zblk, acc.at[pl.ds(s * STRIPE + k * 8, 8)])

    pltpu.sync_copy(dst2d_hbm.at[s], dstloc)
    plsc.subcore_barrier()

    start_gather(0, rowsA, semA)

    @pl.loop(0, NCHUNK // 2)
    def _(it):
        i = it * 2
        start_gather(i + 1, rowsB, semB)
        wait_gather(i, rowsA, semA)
        pltpu.sync_copy(rowsA, acc.at[dstloc.at[i]], add=True)

        @pl.when(i + 2 < NCHUNK)
        def _():
            start_gather(i + 2, rowsA, semA)

        wait_gather(i + 1, rowsB, semB)
        pltpu.sync_copy(rowsB, acc.at[dstloc.at[i + 1]], add=True)

    if NCHUNK % 2 == 1:
        wait_gather(NCHUNK - 1, rowsA, semA)
        pltpu.sync_copy(rowsA, acc.at[dstloc.at[NCHUNK - 1]], add=True)

    plsc.subcore_barrier()

    pltpu.sync_copy(acc.at[pl.ds(s * STRIPE, STRIPE)],
                    out_hbm.at[c, pl.ds(s * STRIPE, STRIPE)])


def _aggregate(src, dst2d, hs2):
    kfn = pl.kernel(
        _agg_body,
        out_type=jax.ShapeDtypeStruct((NC, ACC_ROWS, 128), jnp.float32),
        mesh=_vector_mesh(),
        compiler_params=_sc_compiler_params(),
        scratch_types=[
            pltpu.VMEM((EPW,), jnp.int32),
            pltpu.VMEM((NCHUNK, CHUNK), jnp.int32),
            pltpu.VMEM((CHUNK, 128), jnp.float32),
            pltpu.VMEM((CHUNK, 128), jnp.float32),
            pltpu.VMEM((8, 128), jnp.float32),
            pltpu.VMEM_SHARED((ACC_ROWS, 128), jnp.float32),
            pltpu.SemaphoreType.DMA,
            pltpu.SemaphoreType.DMA,
        ],
    )
    return kfn(src, dst2d, hs2)


def _k1_body(deg_ref, x_ref, w_ref, hs_ref, dis_ref):
    dis = lax.rsqrt(deg_ref[...] + 1.0)
    h = jnp.dot(x_ref[...], w_ref[...], preferred_element_type=jnp.float32)
    hs_ref[...] = (h * dis).reshape(2 * ROWBLK, 128)
    dis_ref[...] = dis


def _k2_body(a0_ref, a1_ref, hs_ref, dis_ref, b_ref, w_ref, hs2_ref):
    dis = dis_ref[...]
    agg = jnp.concatenate([a0_ref[0], a1_ref[0]], axis=1)
    hs = hs_ref[...].reshape(ROWBLK, D)
    t = jnp.maximum(dis * (agg + hs) + b_ref[...], 0.0)
    h2 = jnp.dot(t, w_ref[...], preferred_element_type=jnp.float32)
    hs2_ref[...] = (dis * h2).reshape(2 * ROWBLK, 128)


def _k3_body(a0_ref, a1_ref, hs_ref, dis_ref, b_ref, o_ref):
    agg = jnp.concatenate([a0_ref[0], a1_ref[0]], axis=1)
    hs = hs_ref[...].reshape(ROWBLK, D)
    o_ref[...] = dis_ref[...] * (agg + hs) + b_ref[...]


_NBLK = N // ROWBLK

def _row_spec(width):
    return pl.BlockSpec((ROWBLK, width), lambda i: (i, 0))

def _hs_spec():
    return pl.BlockSpec((2 * ROWBLK, 128), lambda i: (i, 0))

def _agg_spec(core):
    return pl.BlockSpec((1, ROWBLK, 128), lambda i: (core, i, 0))

def _full_spec(r, cols):
    return pl.BlockSpec((r, cols), lambda i: (0, 0))


def _tc_k1(deg, x, W1):
    return pl.pallas_call(
        _k1_body,
        grid=(_NBLK,),
        in_specs=[_row_spec(1), _row_spec(D), _full_spec(D, D)],
        out_specs=[_hs_spec(), _row_spec(1)],
        out_shape=[jax.ShapeDtypeStruct((2 * N, 128), jnp.float32),
                   jax.ShapeDtypeStruct((N, 1), jnp.float32)],
    )(deg, x, W1)


def _tc_k2(agg, hs2, dis, b1, W2):
    return pl.pallas_call(
        _k2_body,
        grid=(_NBLK,),
        in_specs=[_agg_spec(0), _agg_spec(1), _hs_spec(), _row_spec(1),
                  _full_spec(1, D), _full_spec(D, D)],
        out_specs=_hs_spec(),
        out_shape=jax.ShapeDtypeStruct((2 * N, 128), jnp.float32),
    )(agg, agg, hs2, dis, b1, W2)


def _tc_k3(agg, hs2, dis, b2):
    return pl.pallas_call(
        _k3_body,
        grid=(_NBLK,),
        in_specs=[_agg_spec(0), _agg_spec(1), _hs_spec(), _row_spec(1),
                  _full_spec(1, D)],
        out_specs=_row_spec(D),
        out_shape=jax.ShapeDtypeStruct((N, D), jnp.float32),
    )(agg, agg, hs2, dis, b2)


def kernel(x, edge_index, W1, b1, W2, b2):
    src = edge_index[0].astype(jnp.int32)
    dst = edge_index[1].astype(jnp.int32)
    dst2d = dst.reshape(NS, NCHUNK, CHUNK)
    iota80 = jnp.arange(DEG_ROWS, dtype=jnp.int32)
    b1r = b1.reshape(1, D)
    b2r = b2.reshape(1, D)

    deg = _compute_deg(dst, iota80).reshape(-1)[:N].reshape(N, 1)
    hs1, dis = _tc_k1(deg, x, W1)
    agg1 = _aggregate(src, dst2d, hs1)[:, :N, :]
    hs2 = _tc_k2(agg1, hs1, dis, b1r, W2)
    agg2 = _aggregate(src, dst2d, hs2)[:, :N, :]
    return _tc_k3(agg2, hs2, dis, b2r)


# speedup vs baseline: 18.7016x; 1.0629x over previous
import jax
import jax.numpy as jnp
from jax import lax
from jax.experimental import pallas as pl
from jax.experimental.pallas import tpu as pltpu
from jax.experimental.pallas import tpu_sc as plsc

N = 10000
D = 256
E = 160000

NC = 2
NS = 16
ACC_ROWS = 10240
STRIPE = ACC_ROWS // NS
CHUNK = 80
EPW = E // NS
NCHUNK = EPW // CHUNK
DEG_ROWS = 80

ROWBLK = 1000

import dataclasses as _dataclasses
import functools as _functools


@_functools.cache
def _sc_compiler_params():
    cp = pltpu.CompilerParams()
    if "needs_layout_passes" in pltpu.CompilerParams.__dataclass_fields__:
        cp = _dataclasses.replace(cp, needs_layout_passes=False)
    return cp


@_functools.cache
def _vector_mesh():
    return plsc.VectorSubcoreMesh(core_axis_name="core",
                                  subcore_axis_name="subcore",
                                  num_cores=NC, num_subcores=NS)


def _deg_body(edge_hbm, iota_hbm, deg_hbm, dstv, part, iotav, spdeg):
    c = lax.axis_index("core")
    s = lax.axis_index("subcore")
    stripe = 8

    @pl.loop(0, DEG_ROWS)
    def _(i):
        for j in range(8):
            part[i, pl.ds(j * 16, 16)] = jnp.zeros((16,), jnp.float32)

    @pl.when(s < DEG_ROWS // stripe)
    def _():
        pltpu.sync_copy(part.at[pl.ds(0, stripe)],
                        spdeg.at[pl.ds(s * stripe, stripe)])

    pltpu.sync_copy(iota_hbm, iotav)
    pltpu.sync_copy(edge_hbm.at[pl.ds(E + s * EPW, EPW)], dstv)

    @pl.loop(0, EPW // 16)
    def _(i):
        idx = dstv[pl.ds(i * 16, 16)]
        row = lax.shift_right_logical(idx, 7)
        col = jnp.bitwise_and(idx, 127)
        plsc.addupdate_scatter(part, [row, col], jnp.ones((16,), jnp.float32))

    plsc.subcore_barrier()
    pltpu.sync_copy(part, spdeg.at[iotav], add=True)
    plsc.subcore_barrier()

    @pl.when((c == 0) & (s < DEG_ROWS // stripe))
    def _():
        pltpu.sync_copy(spdeg.at[pl.ds(s * stripe, stripe)],
                        deg_hbm.at[pl.ds(s * stripe, stripe)])


def _compute_deg(edges, iota80):
    kfn = pl.kernel(
        _deg_body,
        out_type=jax.ShapeDtypeStruct((DEG_ROWS, 128), jnp.float32),
        mesh=_vector_mesh(),
        compiler_params=_sc_compiler_params(),
        scratch_types=[
            pltpu.VMEM((EPW,), jnp.int32),
            pltpu.VMEM((DEG_ROWS, 128), jnp.float32),
            pltpu.VMEM((DEG_ROWS,), jnp.int32),
            pltpu.VMEM_SHARED((DEG_ROWS, 128), jnp.float32),
        ],
    )
    return kfn(edges, iota80)


def _agg_body(edge_hbm, hs2_hbm, out_hbm,
              srcv, dstloc, rowsA, rowsB, zblk, acc, semA, semB):
    c = lax.axis_index("core")
    s = lax.axis_index("subcore")

    pltpu.sync_copy(edge_hbm.at[pl.ds(s * EPW, EPW)], srcv)

    @pl.loop(0, EPW // 16)
    def _(i):
        v = srcv[pl.ds(i * 16, 16)]
        srcv[pl.ds(i * 16, 16)] = v + v + c

    @pl.loop(0, 8)
    def _(i):
        for j in range(128 // 16):
            zblk[i, pl.ds(j * 16, 16)] = jnp.zeros((16,), jnp.float32)

    def start_gather(i, buf, sem):
        pltpu.async_copy(hs2_hbm.at[srcv.at[pl.ds(i * CHUNK, CHUNK)]], buf, sem)

    def wait_gather(i, buf, sem):
        pltpu.make_async_copy(hs2_hbm.at[srcv.at[pl.ds(i * CHUNK, CHUNK)]],
                              buf, sem).wait()

    @pl.loop(0, STRIPE // 8)
    def _(k):
        pltpu.sync_copy(zblk, acc.at[pl.ds(s * STRIPE + k * 8, 8)])

    pltpu.sync_copy(edge_hbm.at[pl.ds(E + s * EPW, EPW)], dstloc)
    plsc.subcore_barrier()

    start_gather(0, rowsA, semA)

    @pl.loop(0, NCHUNK // 2)
    def _(it):
        i = it * 2
        start_gather(i + 1, rowsB, semB)
        wait_gather(i, rowsA, semA)
        pltpu.sync_copy(rowsA, acc.at[dstloc.at[pl.ds(i * CHUNK, CHUNK)]],
                        add=True)

        @pl.when(i + 2 < NCHUNK)
        def _():
            start_gather(i + 2, rowsA, semA)

        wait_gather(i + 1, rowsB, semB)
        pltpu.sync_copy(rowsB, acc.at[dstloc.at[pl.ds((i + 1) * CHUNK, CHUNK)]],
                        add=True)

    if NCHUNK % 2 == 1:
        wait_gather(NCHUNK - 1, rowsA, semA)
        pltpu.sync_copy(rowsA,
                        acc.at[dstloc.at[pl.ds((NCHUNK - 1) * CHUNK, CHUNK)]],
                        add=True)

    plsc.subcore_barrier()

    pltpu.sync_copy(acc.at[pl.ds(s * STRIPE, STRIPE)],
                    out_hbm.at[c, pl.ds(s * STRIPE, STRIPE)])


def _aggregate(edges, hs2):
    kfn = pl.kernel(
        _agg_body,
        out_type=jax.ShapeDtypeStruct((NC, ACC_ROWS, 128), jnp.float32),
        mesh=_vector_mesh(),
        compiler_params=_sc_compiler_params(),
        scratch_types=[
            pltpu.VMEM((EPW,), jnp.int32),
            pltpu.VMEM((EPW,), jnp.int32),
            pltpu.VMEM((CHUNK, 128), jnp.float32),
            pltpu.VMEM((CHUNK, 128), jnp.float32),
            pltpu.VMEM((8, 128), jnp.float32),
            pltpu.VMEM_SHARED((ACC_ROWS, 128), jnp.float32),
            pltpu.SemaphoreType.DMA,
            pltpu.SemaphoreType.DMA,
        ],
    )
    return kfn(edges, hs2)


def _k1_body(deg_ref, x_ref, w_ref, hs_ref, dis_ref):
    dis = lax.rsqrt(deg_ref[...] + 1.0)
    h = jnp.dot(x_ref[...], w_ref[...], preferred_element_type=jnp.float32)
    hs_ref[...] = (h * dis).reshape(2 * ROWBLK, 128)
    dis_ref[...] = dis


def _k2_body(a0_ref, a1_ref, hs_ref, dis_ref, b_ref, w_ref, hs2_ref):
    dis = dis_ref[...]
    agg = jnp.concatenate([a0_ref[0], a1_ref[0]], axis=1)
    hs = hs_ref[...].reshape(ROWBLK, D)
    t = jnp.maximum(dis * (agg + hs) + b_ref[...], 0.0)
    h2 = jnp.dot(t, w_ref[...], preferred_element_type=jnp.float32)
    hs2_ref[...] = (dis * h2).reshape(2 * ROWBLK, 128)


def _k3_body(a0_ref, a1_ref, hs_ref, dis_ref, b_ref, o_ref):
    agg = jnp.concatenate([a0_ref[0], a1_ref[0]], axis=1)
    hs = hs_ref[...].reshape(ROWBLK, D)
    o_ref[...] = dis_ref[...] * (agg + hs) + b_ref[...]


_NBLK = N // ROWBLK

def _row_spec(width):
    return pl.BlockSpec((ROWBLK, width), lambda i: (i, 0))

def _hs_spec():
    return pl.BlockSpec((2 * ROWBLK, 128), lambda i: (i, 0))

def _agg_spec(core):
    return pl.BlockSpec((1, ROWBLK, 128), lambda i: (core, i, 0))

def _full_spec(r, cols):
    return pl.BlockSpec((r, cols), lambda i: (0, 0))


def _tc_k1(deg, x, W1):
    return pl.pallas_call(
        _k1_body,
        grid=(_NBLK,),
        in_specs=[_row_spec(1), _row_spec(D), _full_spec(D, D)],
        out_specs=[_hs_spec(), _row_spec(1)],
        out_shape=[jax.ShapeDtypeStruct((2 * N, 128), jnp.float32),
                   jax.ShapeDtypeStruct((N, 1), jnp.float32)],
    )(deg, x, W1)


def _tc_k2(agg, hs2, dis, b1, W2):
    return pl.pallas_call(
        _k2_body,
        grid=(_NBLK,),
        in_specs=[_agg_spec(0), _agg_spec(1), _hs_spec(), _row_spec(1),
                  _full_spec(1, D), _full_spec(D, D)],
        out_specs=_hs_spec(),
        out_shape=jax.ShapeDtypeStruct((2 * N, 128), jnp.float32),
    )(agg, agg, hs2, dis, b1, W2)


def _tc_k3(agg, hs2, dis, b2):
    return pl.pallas_call(
        _k3_body,
        grid=(_NBLK,),
        in_specs=[_agg_spec(0), _agg_spec(1), _hs_spec(), _row_spec(1),
                  _full_spec(1, D)],
        out_specs=_row_spec(D),
        out_shape=jax.ShapeDtypeStruct((N, D), jnp.float32),
    )(agg, agg, hs2, dis, b2)


def kernel(x, edge_index, W1, b1, W2, b2):
    edges = edge_index.astype(jnp.int32).reshape(2 * E)
    iota80 = jnp.arange(DEG_ROWS, dtype=jnp.int32)
    b1r = b1.reshape(1, D)
    b2r = b2.reshape(1, D)

    deg = _compute_deg(edges, iota80).reshape(-1)[:N].reshape(N, 1)
    hs1, dis = _tc_k1(deg, x, W1)
    agg1 = _aggregate(edges, hs1)
    hs2 = _tc_k2(agg1, hs1, dis, b1r, W2)
    agg2 = _aggregate(edges, hs2)
    return _tc_k3(agg2, hs2, dis, b2r)
